# Initial kernel scaffold; baseline (speedup 1.0000x reference)
import jax
import jax.numpy as jnp
from jax.experimental import pallas as pl


def kernel(x, embed):
    raise NotImplementedError("write your pallas kernel here")


# speedup vs baseline: 1.2959x; 1.2959x over previous
import jax
import jax.numpy as jnp
from jax.experimental import pallas as pl
from jax.experimental.pallas import tpu as pltpu

NQ = 4
K = 1024


def _pq_body(x_ref, e_ref, qz_ref, idx_ref):
    xb = x_ref[0, 0]
    e = e_ref[0]
    en = e / jnp.clip(jnp.sqrt(jnp.sum(e * e, axis=1, keepdims=True)), 1e-12)
    xn = xb / jnp.clip(jnp.sqrt(jnp.sum(xb * xb, axis=0, keepdims=True)), 1e-12)
    dist_t = jax.lax.dot_general(
        en, xn, (((1,), (0,)), ((), ())), preferred_element_type=jnp.float32)
    idx = jnp.argmax(dist_t, axis=0)
    idx_ref[0, 0, 0] = idx
    one_hot = (jax.lax.broadcasted_iota(jnp.int32, dist_t.shape, 0)
               == idx[None, :]).astype(jnp.float32)
    qz_ref[0, 0] = jax.lax.dot_general(
        e, one_hot, (((0,), (0,)), ((), ())), preferred_element_type=jnp.float32)


def kernel(x, embed):
    B, C, H, W = x.shape
    nq, k, cq = embed.shape
    hw = H * W
    xg = x.reshape(B, nq, cq, hw)

    qz, idx = pl.pallas_call(
        _pq_body,
        grid=(nq, B),
        in_specs=[
            pl.BlockSpec((1, 1, cq, hw), lambda q, b: (b, q, 0, 0)),
            pl.BlockSpec((1, k, cq), lambda q, b: (q, 0, 0)),
        ],
        out_specs=[
            pl.BlockSpec((1, 1, cq, hw), lambda q, b: (b, q, 0, 0)),
            pl.BlockSpec((1, 1, 1, hw), lambda q, b: (b, q, 0, 0)),
        ],
        out_shape=[
            jax.ShapeDtypeStruct((B, nq, cq, hw), jnp.float32),
            jax.ShapeDtypeStruct((B, nq, 1, hw), jnp.int32),
        ],
        compiler_params=pltpu.CompilerParams(
            dimension_semantics=("arbitrary", "arbitrary")),
    )(xg, embed)

    quantized = qz.reshape(B, C, H, W)
    encoding = idx.reshape(B, nq * H, W)
    vq_loss = jnp.zeros((1,), dtype=jnp.float32)
    return quantized, encoding, vq_loss
